# baseline (speedup 1.0000x reference)
import functools
import math

import jax
import jax.numpy as jnp
from jax import lax
from jax.experimental import pallas as pl
from jax.experimental.pallas import tpu as pltpu
from jax.experimental.pallas import tpu_sc as plsc

D_MODEL = 64
SCALE = math.sqrt(D_MODEL)

NC = 2
NS = 16
NW = NC * NS

SB = 128
NBUF = 2
LANES = 16


def _emb_body(x4_hbm, tab2_hbm, out_hbm, idx_v, gidx,
              gbufs, obufs, gsems, ssems):
    wid = lax.axis_index("s") * NC + lax.axis_index("c")
    ntok = x4_hbm.shape[0] * x4_hbm.shape[2]
    seq0 = wid * SB

    pltpu.sync_copy(x4_hbm.at[:, wid], idx_v)

    def start_gather(b, t):
        rb = t // 8
        rr = t % 8
        for k in range(SB // LANES):
            sl = pl.ds(k * LANES, LANES)
            gidx[b, sl] = lax.shift_right_logical(idx_v[rb, rr, sl], 1)
        pltpu.async_copy(tab2_hbm.at[gidx.at[b]], gbufs[b], gsems[b])

    def start_store(b, t):
        pltpu.async_copy(obufs[b], out_hbm.at[pl.ds(seq0, SB), t], ssems[b])

    def wait_store(b, t):
        pltpu.make_async_copy(obufs[b], out_hbm.at[pl.ds(seq0, SB), t],
                              ssems[b]).wait()

    for b in range(NBUF - 1):
        start_gather(b, b)

    def round_body(r):
        for b in range(NBUF):
            t = r * NBUF + b
            rb = t // 8
            rr = t % 8

            pltpu.make_async_copy(tab2_hbm.at[gidx.at[b]], gbufs[b],
                                  gsems[b]).wait()

            iot = lax.iota(jnp.int32, LANES)
            rbv = jnp.broadcast_to(rb, (LANES,)).astype(jnp.int32)
            rrv = jnp.broadcast_to(rr, (LANES,)).astype(jnp.int32)

            def select_scale_row(row, _):
                rv = jnp.broadcast_to(row, (LANES,)).astype(jnp.int32)
                v = plsc.load_gather(idx_v, [rbv, rrv, rv])
                off = (v & 1) * D_MODEL
                for c in range(D_MODEL // LANES):
                    col = off + (c * LANES) + iot
                    val = plsc.load_gather(gbufs[b], [rv, col])
                    obufs[b][row, pl.ds(c * LANES, LANES)] = val * SCALE
                return 0

            lax.fori_loop(0, SB, select_scale_row, 0, unroll=2)

            start_store(b, t)

            bp = (b - 1) % NBUF
            tp = t - 1

            @pl.when(tp >= 0)
            def _():
                wait_store(bp, tp)

            @pl.when(tp + NBUF < ntok)
            def _():
                start_gather(bp, tp + NBUF)

    pl.loop(0, ntok // NBUF)(round_body)

    wait_store((ntok - 1) % NBUF, ntok - 1)


@jax.jit
def _emb_call(x4, tab2):
    ntok = x4.shape[0] * x4.shape[2]
    nseq = x4.shape[1] * x4.shape[3]
    mesh = plsc.VectorSubcoreMesh(core_axis_name="c", subcore_axis_name="s",
                                  num_cores=NC, num_subcores=NS)
    scratch = (
        [pltpu.VMEM((x4.shape[0], x4.shape[2], SB), jnp.int32)]
        + [pltpu.VMEM((NBUF, SB), jnp.int32)]
        + [[pltpu.VMEM((SB, 2 * D_MODEL), jnp.float32) for _ in range(NBUF)]]
        + [[pltpu.VMEM((SB, D_MODEL), jnp.float32) for _ in range(NBUF)]]
        + [[pltpu.SemaphoreType.DMA for _ in range(NBUF)]]
        + [[pltpu.SemaphoreType.DMA for _ in range(NBUF)]]
    )
    kern = pl.kernel(
        _emb_body,
        out_type=jax.ShapeDtypeStruct((nseq, ntok, D_MODEL), jnp.float32),
        mesh=mesh,
        scratch_types=scratch,
        compiler_params=pltpu.CompilerParams(needs_layout_passes=False),
    )
    return kern(x4, tab2)


def kernel(x, table):
    nseq, ntok = x.shape
    x4 = x.T.reshape(ntok // 8, 8, nseq // SB, SB).transpose(0, 2, 1, 3)
    tab2 = table.reshape(-1, 2 * D_MODEL)
    return _emb_call(x4, tab2)
